# baseline (speedup 1.0000x reference)
import functools

import jax
import jax.numpy as jnp
from jax import lax
from jax.experimental import pallas as pl
from jax.experimental.pallas import tpu as pltpu
from jax.experimental.pallas import tpu_sc as plsc

N_VOCAB = 100000
WINDOW = 20
EMB = 32
HIDDEN = 128
BATCH = 1024

NUM_IDX = BATCH * WINDOW
SC_CORES = 2
SC_SUBCORES = 16
NUM_WORKERS = SC_CORES * SC_SUBCORES
B_PER_W = NUM_IDX // NUM_WORKERS

TILE_V = 2048
GRID_V = pl.cdiv(N_VOCAB, TILE_V)


def _sc_gather(C, idx_flat):
    mesh = plsc.VectorSubcoreMesh(core_axis_name="c", subcore_axis_name="s")

    @functools.partial(
        pl.kernel,
        mesh=mesh,
        out_type=jax.ShapeDtypeStruct((NUM_IDX, EMB), jnp.float32),
        scratch_types=[
            pltpu.VMEM((B_PER_W,), jnp.int32),
            pltpu.VMEM((B_PER_W, EMB), jnp.float32),
            pltpu.SemaphoreType.DMA,
        ],
        compiler_params=pltpu.CompilerParams(use_tc_tiling_on_sc=False),
    )
    def gather_kernel(table_hbm, idx_hbm, out_hbm, idx_v, rows_v, sem):
        wid = lax.axis_index("s") * SC_CORES + lax.axis_index("c")
        base = wid * B_PER_W
        pltpu.sync_copy(idx_hbm.at[pl.ds(base, B_PER_W)], idx_v)
        pltpu.async_copy(table_hbm.at[idx_v], rows_v, sem).wait()
        pltpu.sync_copy(rows_v, out_hbm.at[pl.ds(base, B_PER_W)])

    return gather_kernel(C, idx_flat)


def _h_body(flat_ref, w1_ref, b1_ref, h_ref):
    pre = lax.dot_general(
        flat_ref[...], w1_ref[...],
        (((1,), (1,)), ((), ())),
        preferred_element_type=jnp.float32,
    )
    h_ref[...] = jnp.tanh(pre + b1_ref[...])


def _logits_body(h_ref, w2_ref, b2_ref, out_ref):
    h = h_ref[...].astype(jnp.bfloat16)
    w2 = w2_ref[...].astype(jnp.bfloat16)
    acc = lax.dot_general(
        h, w2,
        (((1,), (1,)), ((), ())),
        preferred_element_type=jnp.float32,
    )
    out_ref[...] = acc + b2_ref[...]


def kernel(x, C, W1, b1, W2, b2):
    idx_flat = x.reshape(NUM_IDX).astype(jnp.int32)
    emb = _sc_gather(C, idx_flat)
    flat = emb.reshape(BATCH, WINDOW * EMB)

    b1_2d = b1.reshape(1, HIDDEN)
    b2_2d = b2.reshape(1, N_VOCAB)

    h = pl.pallas_call(
        _h_body,
        out_shape=jax.ShapeDtypeStruct((BATCH, HIDDEN), jnp.float32),
    )(flat, W1, b1_2d)

    logits = pl.pallas_call(
        _logits_body,
        grid=(GRID_V,),
        in_specs=[
            pl.BlockSpec((BATCH, HIDDEN), lambda i: (0, 0)),
            pl.BlockSpec((TILE_V, HIDDEN), lambda i: (i, 0)),
            pl.BlockSpec((1, TILE_V), lambda i: (0, i)),
        ],
        out_specs=pl.BlockSpec((BATCH, TILE_V), lambda i: (0, i)),
        out_shape=jax.ShapeDtypeStruct((BATCH, N_VOCAB), jnp.float32),
        compiler_params=pltpu.CompilerParams(
            dimension_semantics=("parallel",),
        ),
    )(h, W2, b2_2d)

    return logits


# speedup vs baseline: 2.0681x; 2.0681x over previous
import functools

import jax
import jax.numpy as jnp
from jax import lax
from jax.experimental import pallas as pl
from jax.experimental.pallas import tpu as pltpu
from jax.experimental.pallas import tpu_sc as plsc

N_VOCAB = 100000
WINDOW = 20
EMB = 32
HIDDEN = 128
BATCH = 1024

NUM_IDX = BATCH * WINDOW
SC_CORES = 2
SC_SUBCORES = 16
NUM_WORKERS = SC_CORES * SC_SUBCORES
B_PER_W = NUM_IDX // NUM_WORKERS

TILE_V = 2048
GRID_V = pl.cdiv(N_VOCAB, TILE_V)


def _sc_gather(C, idx_flat):
    mesh = plsc.VectorSubcoreMesh(core_axis_name="c", subcore_axis_name="s")

    @functools.partial(
        pl.kernel,
        mesh=mesh,
        out_type=jax.ShapeDtypeStruct((NUM_IDX, EMB), jnp.float32),
        scratch_types=[
            pltpu.VMEM((B_PER_W,), jnp.int32),
            pltpu.VMEM((B_PER_W, EMB), jnp.float32),
            pltpu.SemaphoreType.DMA,
        ],
        compiler_params=pltpu.CompilerParams(use_tc_tiling_on_sc=False),
    )
    def gather_kernel(table_hbm, idx_hbm, out_hbm, idx_v, rows_v, sem):
        wid = lax.axis_index("s") * SC_CORES + lax.axis_index("c")
        base = wid * B_PER_W
        pltpu.sync_copy(idx_hbm.at[pl.ds(base, B_PER_W)], idx_v)
        pltpu.async_copy(table_hbm.at[idx_v], rows_v, sem).wait()
        pltpu.sync_copy(rows_v, out_hbm.at[pl.ds(base, B_PER_W)])

    return gather_kernel(C, idx_flat)


def _h_body(flat_ref, w1_ref, b1_ref, h_ref):
    pre = lax.dot_general(
        flat_ref[...], w1_ref[...],
        (((1,), (1,)), ((), ())),
        preferred_element_type=jnp.float32,
    )
    h_ref[...] = jnp.tanh(pre + b1_ref[...])


def _logits_body(h_ref, w2_ref, b2_ref, out_ref):
    h = h_ref[...].astype(jnp.bfloat16)
    w2 = w2_ref[...].astype(jnp.bfloat16)
    acc = lax.dot_general(
        w2, h,
        (((1,), (1,)), ((), ())),
        preferred_element_type=jnp.float32,
    )
    out_ref[...] = acc + b2_ref[...]


def kernel(x, C, W1, b1, W2, b2):
    idx_flat = x.reshape(NUM_IDX).astype(jnp.int32)
    emb = _sc_gather(C, idx_flat)
    flat = emb.reshape(BATCH, WINDOW * EMB)

    b1_2d = b1.reshape(1, HIDDEN)
    b2_col = b2.reshape(N_VOCAB, 1)

    h = pl.pallas_call(
        _h_body,
        out_shape=jax.ShapeDtypeStruct((BATCH, HIDDEN), jnp.float32),
    )(flat, W1, b1_2d)

    logits_t = pl.pallas_call(
        _logits_body,
        grid=(GRID_V,),
        in_specs=[
            pl.BlockSpec((BATCH, HIDDEN), lambda i: (0, 0)),
            pl.BlockSpec((TILE_V, HIDDEN), lambda i: (i, 0)),
            pl.BlockSpec((TILE_V, 1), lambda i: (i, 0)),
        ],
        out_specs=pl.BlockSpec((TILE_V, BATCH), lambda i: (i, 0)),
        out_shape=jax.ShapeDtypeStruct((N_VOCAB, BATCH), jnp.float32),
        compiler_params=pltpu.CompilerParams(
            dimension_semantics=("parallel",),
        ),
    )(h, W2, b2_col)

    return logits_t.T


# speedup vs baseline: 2.5890x; 1.2519x over previous
import functools

import jax
import jax.numpy as jnp
from jax import lax
from jax.experimental import pallas as pl
from jax.experimental.pallas import tpu as pltpu
from jax.experimental.pallas import tpu_sc as plsc

N_VOCAB = 100000
WINDOW = 20
EMB = 32
HIDDEN = 128
BATCH = 1024

NUM_IDX = BATCH * WINDOW
SC_CORES = 2
SC_SUBCORES = 16
NUM_WORKERS = SC_CORES * SC_SUBCORES
B_PER_W = NUM_IDX // NUM_WORKERS

TILE_V = 2048
GRID_V = pl.cdiv(N_VOCAB, TILE_V)


def _sc_gather(C, idx_flat):
    mesh = plsc.VectorSubcoreMesh(core_axis_name="c", subcore_axis_name="s")

    @functools.partial(
        pl.kernel,
        mesh=mesh,
        out_type=jax.ShapeDtypeStruct((NUM_IDX, EMB), jnp.float32),
        scratch_types=[
            pltpu.VMEM((B_PER_W,), jnp.int32),
            pltpu.VMEM((B_PER_W, EMB), jnp.float32),
            pltpu.SemaphoreType.DMA,
        ],
        compiler_params=pltpu.CompilerParams(use_tc_tiling_on_sc=False),
    )
    def gather_kernel(table_hbm, idx_hbm, out_hbm, idx_v, rows_v, sem):
        wid = lax.axis_index("s") * SC_CORES + lax.axis_index("c")
        base = wid * B_PER_W
        pltpu.sync_copy(idx_hbm.at[pl.ds(base, B_PER_W)], idx_v)
        pltpu.async_copy(table_hbm.at[idx_v], rows_v, sem).wait()
        pltpu.sync_copy(rows_v, out_hbm.at[pl.ds(base, B_PER_W)])

    return gather_kernel(C, idx_flat)


def _h_body(flat_ref, w1_ref, b1_ref, ht_ref):
    pre = lax.dot_general(
        w1_ref[...], flat_ref[...],
        (((1,), (1,)), ((), ())),
        preferred_element_type=jnp.float32,
    )
    ht_ref[...] = jnp.tanh(pre + b1_ref[...])


def _logits_body(ht_ref, w2_ref, b2_ref, out_ref):
    ht = ht_ref[...].astype(jnp.bfloat16)
    w2 = w2_ref[...].astype(jnp.bfloat16)
    acc = lax.dot_general(
        w2, ht,
        (((1,), (0,)), ((), ())),
        preferred_element_type=jnp.float32,
    )
    out_ref[...] = acc + jnp.transpose(b2_ref[...])


def kernel(x, C, W1, b1, W2, b2):
    idx_flat = x.reshape(NUM_IDX).astype(jnp.int32)
    emb = _sc_gather(C, idx_flat)
    flat = emb.reshape(BATCH, WINDOW * EMB)

    b1_col = b1.reshape(HIDDEN, 1)
    b2_row = b2.reshape(1, N_VOCAB)

    ht = pl.pallas_call(
        _h_body,
        out_shape=jax.ShapeDtypeStruct((HIDDEN, BATCH), jnp.float32),
    )(flat, W1, b1_col)

    logits_t = pl.pallas_call(
        _logits_body,
        grid=(GRID_V,),
        in_specs=[
            pl.BlockSpec((HIDDEN, BATCH), lambda i: (0, 0)),
            pl.BlockSpec((TILE_V, HIDDEN), lambda i: (i, 0)),
            pl.BlockSpec((1, TILE_V), lambda i: (0, i)),
        ],
        out_specs=pl.BlockSpec((TILE_V, BATCH), lambda i: (i, 0)),
        out_shape=jax.ShapeDtypeStruct((N_VOCAB, BATCH), jnp.float32),
        compiler_params=pltpu.CompilerParams(
            dimension_semantics=("parallel",),
        ),
    )(ht, W2, b2_row)

    return logits_t.T


# speedup vs baseline: 2.6223x; 1.0129x over previous
import functools

import jax
import jax.numpy as jnp
from jax import lax
from jax.experimental import pallas as pl
from jax.experimental.pallas import tpu as pltpu
from jax.experimental.pallas import tpu_sc as plsc

N_VOCAB = 100000
WINDOW = 20
EMB = 32
HIDDEN = 128
BATCH = 1024

NUM_IDX = BATCH * WINDOW
SC_CORES = 2
SC_SUBCORES = 16
NUM_WORKERS = SC_CORES * SC_SUBCORES
B_PER_W = NUM_IDX // NUM_WORKERS

TILE_V = 4096
GRID_V = pl.cdiv(N_VOCAB, TILE_V)


def _sc_gather(C, idx_flat):
    mesh = plsc.VectorSubcoreMesh(core_axis_name="c", subcore_axis_name="s")

    @functools.partial(
        pl.kernel,
        mesh=mesh,
        out_type=jax.ShapeDtypeStruct((NUM_IDX, EMB), jnp.float32),
        scratch_types=[
            pltpu.VMEM((B_PER_W,), jnp.int32),
            pltpu.VMEM((B_PER_W, EMB), jnp.float32),
            pltpu.SemaphoreType.DMA,
        ],
        compiler_params=pltpu.CompilerParams(use_tc_tiling_on_sc=False),
    )
    def gather_kernel(table_hbm, idx_hbm, out_hbm, idx_v, rows_v, sem):
        wid = lax.axis_index("s") * SC_CORES + lax.axis_index("c")
        base = wid * B_PER_W
        pltpu.sync_copy(idx_hbm.at[pl.ds(base, B_PER_W)], idx_v)
        pltpu.async_copy(table_hbm.at[idx_v], rows_v, sem).wait()
        pltpu.sync_copy(rows_v, out_hbm.at[pl.ds(base, B_PER_W)])

    return gather_kernel(C, idx_flat)


def _h_body(flat_ref, w1_ref, b1_ref, ht_ref):
    pre = lax.dot_general(
        w1_ref[...], flat_ref[...],
        (((1,), (1,)), ((), ())),
        preferred_element_type=jnp.float32,
    )
    ht_ref[...] = jnp.tanh(pre + b1_ref[...])


def _logits_body(ht_ref, w2_ref, b2_ref, out_ref):
    ht = ht_ref[...].astype(jnp.bfloat16)
    w2 = w2_ref[...].astype(jnp.bfloat16)
    acc = lax.dot_general(
        w2, ht,
        (((1,), (0,)), ((), ())),
        preferred_element_type=jnp.float32,
    )
    out_ref[...] = acc + jnp.transpose(b2_ref[...])


def kernel(x, C, W1, b1, W2, b2):
    idx_flat = x.reshape(NUM_IDX).astype(jnp.int32)
    emb = _sc_gather(C, idx_flat)
    flat = emb.reshape(BATCH, WINDOW * EMB)

    b1_col = b1.reshape(HIDDEN, 1)
    b2_row = b2.reshape(1, N_VOCAB)

    ht = pl.pallas_call(
        _h_body,
        out_shape=jax.ShapeDtypeStruct((HIDDEN, BATCH), jnp.float32),
    )(flat, W1, b1_col)

    logits_t = pl.pallas_call(
        _logits_body,
        grid=(GRID_V,),
        in_specs=[
            pl.BlockSpec((HIDDEN, BATCH), lambda i: (0, 0)),
            pl.BlockSpec((TILE_V, HIDDEN), lambda i: (i, 0)),
            pl.BlockSpec((1, TILE_V), lambda i: (0, i)),
        ],
        out_specs=pl.BlockSpec((TILE_V, BATCH), lambda i: (i, 0)),
        out_shape=jax.ShapeDtypeStruct((N_VOCAB, BATCH), jnp.float32),
        compiler_params=pltpu.CompilerParams(
            dimension_semantics=("parallel",),
        ),
    )(ht, W2, b2_row)

    return logits_t.T


# speedup vs baseline: 2.6229x; 1.0002x over previous
import functools

import jax
import jax.numpy as jnp
from jax import lax
from jax.experimental import pallas as pl
from jax.experimental.pallas import tpu as pltpu
from jax.experimental.pallas import tpu_sc as plsc

N_VOCAB = 100000
WINDOW = 20
EMB = 32
HIDDEN = 128
BATCH = 1024

NUM_IDX = BATCH * WINDOW
SC_CORES = 2
SC_SUBCORES = 16
NUM_WORKERS = SC_CORES * SC_SUBCORES
B_PER_W = NUM_IDX // NUM_WORKERS

TILE_V = 4096
GRID_V = pl.cdiv(N_VOCAB, TILE_V)


def _sc_gather(table, idx_flat):
    mesh = plsc.VectorSubcoreMesh(core_axis_name="c", subcore_axis_name="s")

    @functools.partial(
        pl.kernel,
        mesh=mesh,
        out_type=jax.ShapeDtypeStruct((NUM_IDX, EMB), jnp.float32),
        scratch_types=[
            pltpu.VMEM((B_PER_W,), jnp.int32),
            pltpu.VMEM((B_PER_W, EMB), jnp.float32),
            pltpu.SemaphoreType.DMA,
        ],
        compiler_params=pltpu.CompilerParams(use_tc_tiling_on_sc=False),
    )
    def gather_kernel(table_hbm, idx_hbm, out_hbm, idx_v, rows_v, sem):
        wid = lax.axis_index("s") * SC_CORES + lax.axis_index("c")
        base = wid * B_PER_W
        pltpu.sync_copy(idx_hbm.at[pl.ds(base, B_PER_W)], idx_v)
        pltpu.async_copy(table_hbm.at[idx_v], rows_v, sem).wait()
        pltpu.sync_copy(rows_v, out_hbm.at[pl.ds(base, B_PER_W)])

    return gather_kernel(table, idx_flat)


def _h_body(flat_ref, w1_ref, b1_ref, ht_ref):
    pre = lax.dot_general(
        w1_ref[...], flat_ref[...],
        (((1,), (1,)), ((), ())),
        preferred_element_type=jnp.float32,
    )
    ht_ref[...] = jnp.tanh(pre + b1_ref[...]).astype(jnp.bfloat16)


def _logits_body(ht_ref, w2_ref, b2_ref, out_ref):
    ht = ht_ref[...]
    w2 = w2_ref[...].astype(jnp.bfloat16)
    acc = lax.dot_general(
        w2, ht,
        (((1,), (0,)), ((), ())),
        preferred_element_type=jnp.float32,
    )
    out_ref[...] = acc + jnp.transpose(b2_ref[...])


def kernel(x, C, W1, b1, W2, b2):
    idx_flat = x.reshape(NUM_IDX).astype(jnp.int32)
    emb = _sc_gather(C, idx_flat)
    flat = emb.reshape(BATCH, WINDOW * EMB)

    b1_col = b1.reshape(HIDDEN, 1)
    b2_row = b2.reshape(1, N_VOCAB)

    ht = pl.pallas_call(
        _h_body,
        out_shape=jax.ShapeDtypeStruct((HIDDEN, BATCH), jnp.bfloat16),
    )(flat, W1, b1_col)

    logits_t = pl.pallas_call(
        _logits_body,
        grid=(GRID_V,),
        in_specs=[
            pl.BlockSpec((HIDDEN, BATCH), lambda i: (0, 0)),
            pl.BlockSpec((TILE_V, HIDDEN), lambda i: (i, 0)),
            pl.BlockSpec((1, TILE_V), lambda i: (0, i)),
        ],
        out_specs=pl.BlockSpec((TILE_V, BATCH), lambda i: (i, 0)),
        out_shape=jax.ShapeDtypeStruct((N_VOCAB, BATCH), jnp.float32),
        compiler_params=pltpu.CompilerParams(
            dimension_semantics=("parallel",),
        ),
    )(ht, W2, b2_row)

    return logits_t.T


# speedup vs baseline: 2.6465x; 1.0090x over previous
import functools

import jax
import jax.numpy as jnp
from jax import lax
from jax.experimental import pallas as pl
from jax.experimental.pallas import tpu as pltpu
from jax.experimental.pallas import tpu_sc as plsc

N_VOCAB = 100000
WINDOW = 20
EMB = 32
HIDDEN = 128
BATCH = 1024

NUM_IDX = BATCH * WINDOW
SC_CORES = 2
SC_SUBCORES = 16
NUM_WORKERS = SC_CORES * SC_SUBCORES
B_PER_W = NUM_IDX // NUM_WORKERS

TILE_V = 4096
GRID_V = pl.cdiv(N_VOCAB, TILE_V)


def _sc_gather(table, idx_flat):
    mesh = plsc.VectorSubcoreMesh(core_axis_name="c", subcore_axis_name="s")

    @functools.partial(
        pl.kernel,
        mesh=mesh,
        out_type=jax.ShapeDtypeStruct((NUM_IDX, EMB), jnp.float32),
        scratch_types=[
            pltpu.VMEM((B_PER_W,), jnp.int32),
            pltpu.VMEM((B_PER_W, EMB), jnp.float32),
            pltpu.SemaphoreType.DMA,
        ],
        compiler_params=pltpu.CompilerParams(use_tc_tiling_on_sc=False),
    )
    def gather_kernel(table_hbm, idx_hbm, out_hbm, idx_v, rows_v, sem):
        wid = lax.axis_index("s") * SC_CORES + lax.axis_index("c")
        base = wid * B_PER_W
        pltpu.sync_copy(idx_hbm.at[pl.ds(base, B_PER_W)], idx_v)
        pltpu.async_copy(table_hbm.at[idx_v], rows_v, sem).wait()
        pltpu.sync_copy(rows_v, out_hbm.at[pl.ds(base, B_PER_W)])

    return gather_kernel(table, idx_flat)


def _logits_body(flat_ref, w1_ref, b1_ref, w2_ref, b2_ref, out_ref, ht_s):
    @pl.when(pl.program_id(0) == 0)
    def _():
        pre = lax.dot_general(
            w1_ref[...], flat_ref[...],
            (((1,), (1,)), ((), ())),
            preferred_element_type=jnp.float32,
        )
        ht_s[...] = jnp.tanh(pre + b1_ref[...]).astype(jnp.bfloat16)

    w2 = w2_ref[...].astype(jnp.bfloat16)
    acc = lax.dot_general(
        w2, ht_s[...],
        (((1,), (0,)), ((), ())),
        preferred_element_type=jnp.float32,
    )
    out_ref[...] = acc + jnp.transpose(b2_ref[...])


def kernel(x, C, W1, b1, W2, b2):
    idx_flat = x.reshape(NUM_IDX).astype(jnp.int32)
    emb = _sc_gather(C, idx_flat)
    flat = emb.reshape(BATCH, WINDOW * EMB)

    b1_col = b1.reshape(HIDDEN, 1)
    b2_row = b2.reshape(1, N_VOCAB)

    logits_t = pl.pallas_call(
        _logits_body,
        grid=(GRID_V,),
        in_specs=[
            pl.BlockSpec((BATCH, WINDOW * EMB), lambda i: (0, 0)),
            pl.BlockSpec((HIDDEN, WINDOW * EMB), lambda i: (0, 0)),
            pl.BlockSpec((HIDDEN, 1), lambda i: (0, 0)),
            pl.BlockSpec((TILE_V, HIDDEN), lambda i: (i, 0)),
            pl.BlockSpec((1, TILE_V), lambda i: (0, i)),
        ],
        out_specs=pl.BlockSpec((TILE_V, BATCH), lambda i: (i, 0)),
        out_shape=jax.ShapeDtypeStruct((N_VOCAB, BATCH), jnp.float32),
        scratch_shapes=[pltpu.VMEM((HIDDEN, BATCH), jnp.bfloat16)],
        compiler_params=pltpu.CompilerParams(
            dimension_semantics=("arbitrary",),
        ),
    )(flat, W1, b1_col, W2, b2_row)

    return logits_t.T


# speedup vs baseline: 2.6505x; 1.0015x over previous
import functools

import jax
import jax.numpy as jnp
from jax import lax
from jax.experimental import pallas as pl
from jax.experimental.pallas import tpu as pltpu
from jax.experimental.pallas import tpu_sc as plsc

N_VOCAB = 100000
WINDOW = 20
EMB = 32
HIDDEN = 128
BATCH = 1024

NUM_IDX = BATCH * WINDOW
SC_CORES = 2
SC_SUBCORES = 16
NUM_WORKERS = SC_CORES * SC_SUBCORES
B_PER_W = NUM_IDX // NUM_WORKERS

TILE_V = 5120
GRID_V = pl.cdiv(N_VOCAB, TILE_V)


def _sc_gather(table, idx_flat):
    mesh = plsc.VectorSubcoreMesh(core_axis_name="c", subcore_axis_name="s")

    @functools.partial(
        pl.kernel,
        mesh=mesh,
        out_type=jax.ShapeDtypeStruct((NUM_IDX, EMB), jnp.float32),
        scratch_types=[
            pltpu.VMEM((B_PER_W,), jnp.int32),
            pltpu.VMEM((B_PER_W, EMB), jnp.float32),
            pltpu.SemaphoreType.DMA,
        ],
        compiler_params=pltpu.CompilerParams(use_tc_tiling_on_sc=False),
    )
    def gather_kernel(table_hbm, idx_hbm, out_hbm, idx_v, rows_v, sem):
        wid = lax.axis_index("s") * SC_CORES + lax.axis_index("c")
        base = wid * B_PER_W
        pltpu.sync_copy(idx_hbm.at[pl.ds(base, B_PER_W)], idx_v)
        pltpu.async_copy(table_hbm.at[idx_v], rows_v, sem).wait()
        pltpu.sync_copy(rows_v, out_hbm.at[pl.ds(base, B_PER_W)])

    return gather_kernel(table, idx_flat)


def _logits_body(flat_ref, w1_ref, b1_ref, w2_ref, b2_ref, out_ref, ht_s):
    @pl.when(pl.program_id(0) == 0)
    def _():
        pre = lax.dot_general(
            w1_ref[...], flat_ref[...],
            (((1,), (1,)), ((), ())),
            preferred_element_type=jnp.float32,
        )
        ht_s[...] = jnp.tanh(pre + b1_ref[...]).astype(jnp.bfloat16)

    w2 = w2_ref[...].astype(jnp.bfloat16)
    acc = lax.dot_general(
        w2, ht_s[...],
        (((1,), (0,)), ((), ())),
        preferred_element_type=jnp.float32,
    )
    out_ref[...] = acc + jnp.transpose(b2_ref[...])


def kernel(x, C, W1, b1, W2, b2):
    idx_flat = x.reshape(NUM_IDX).astype(jnp.int32)
    emb = _sc_gather(C, idx_flat)
    flat = emb.reshape(BATCH, WINDOW * EMB)

    b1_col = b1.reshape(HIDDEN, 1)
    b2_row = b2.reshape(1, N_VOCAB)

    logits_t = pl.pallas_call(
        _logits_body,
        grid=(GRID_V,),
        in_specs=[
            pl.BlockSpec((BATCH, WINDOW * EMB), lambda i: (0, 0)),
            pl.BlockSpec((HIDDEN, WINDOW * EMB), lambda i: (0, 0)),
            pl.BlockSpec((HIDDEN, 1), lambda i: (0, 0)),
            pl.BlockSpec((TILE_V, HIDDEN), lambda i: (i, 0)),
            pl.BlockSpec((1, TILE_V), lambda i: (0, i)),
        ],
        out_specs=pl.BlockSpec((TILE_V, BATCH), lambda i: (i, 0)),
        out_shape=jax.ShapeDtypeStruct((N_VOCAB, BATCH), jnp.float32),
        scratch_shapes=[pltpu.VMEM((HIDDEN, BATCH), jnp.bfloat16)],
        compiler_params=pltpu.CompilerParams(
            dimension_semantics=("arbitrary",),
        ),
    )(flat, W1, b1_col, W2, b2_row)

    return logits_t.T
